# baseline (speedup 1.0000x reference)
import functools

import jax
import jax.numpy as jnp
from jax import lax
from jax.experimental import pallas as pl
from jax.experimental.pallas import tpu as pltpu
from jax.experimental.pallas import tpu_sc as plsc

BATCH = 4096
DIM = 10000
LANES = 16
NUM_WORKERS = 32
ROWS_PER_WORKER = BATCH // NUM_WORKERS
NSEG = DIM // LANES


def _sc_permute(inp, sel):
    mesh = plsc.VectorSubcoreMesh(core_axis_name="c", subcore_axis_name="s")

    @functools.partial(
        pl.kernel,
        mesh=mesh,
        out_type=jax.ShapeDtypeStruct((BATCH, DIM), jnp.float32),
        scratch_types=[
            pltpu.VMEM((DIM,), jnp.int32),
            pltpu.VMEM((DIM,), jnp.int32),
            pltpu.VMEM((DIM,), jnp.float32),
            pltpu.VMEM((DIM,), jnp.float32),
        ],
        compiler_params=pltpu.CompilerParams(needs_layout_passes=False),
    )
    def k(in_hbm, sel_hbm, out_hbm, sel_v, c_v, row_v, orow_v):
        cid = lax.axis_index("c")
        sid = lax.axis_index("s")
        wid = sid * 2 + cid
        base = wid * ROWS_PER_WORKER

        pltpu.sync_copy(sel_hbm, sel_v)

        def compose(j, carry):
            off = pl.multiple_of(j * LANES, LANES)
            seg = sel_v[pl.ds(off, LANES)]
            c_v[pl.ds(off, LANES)] = plsc.load_gather(sel_v, [seg])
            return carry

        lax.fori_loop(0, NSEG, compose, 0)

        def row_loop(r, carry):
            row = base + r
            pltpu.sync_copy(in_hbm.at[row], row_v)

            def gath(j, c2):
                off = pl.multiple_of(j * LANES, LANES)
                idx = c_v[pl.ds(off, LANES)]
                orow_v[pl.ds(off, LANES)] = plsc.load_gather(row_v, [idx])
                return c2

            lax.fori_loop(0, NSEG, gath, 0)
            pltpu.sync_copy(orow_v, out_hbm.at[row])
            return carry

        lax.fori_loop(0, ROWS_PER_WORKER, row_loop, 0)

    return k(inp, sel)


def kernel(input, forward_indices, backward_indices, shifts):
    ident = jnp.arange(DIM, dtype=jnp.int32)
    sel = jnp.where(
        shifts > 0,
        forward_indices,
        jnp.where(shifts < 0, backward_indices, ident),
    )
    return _sc_permute(input, sel)


# speedup vs baseline: 1.5559x; 1.5559x over previous
import functools

import jax
import jax.numpy as jnp
from jax import lax
from jax.experimental import pallas as pl
from jax.experimental.pallas import tpu as pltpu
from jax.experimental.pallas import tpu_sc as plsc

BATCH = 4096
DIM = 10000
LANES = 16
NUM_WORKERS = 32
ROWS_PER_WORKER = BATCH // NUM_WORKERS
NSEG = DIM // LANES
R = 2
RN = R * DIM
CHUNKS = ROWS_PER_WORKER // R
TPAIR = CHUNKS // 2


def _sc_permute(inp_flat, sel):
    mesh = plsc.VectorSubcoreMesh(core_axis_name="c", subcore_axis_name="s")

    @functools.partial(
        pl.kernel,
        mesh=mesh,
        out_type=jax.ShapeDtypeStruct((BATCH * DIM,), jnp.float32),
        scratch_types=[
            pltpu.VMEM((DIM,), jnp.int32),
            pltpu.VMEM((DIM,), jnp.int32),
            pltpu.VMEM((RN,), jnp.float32),
            pltpu.VMEM((RN,), jnp.float32),
            pltpu.VMEM((RN,), jnp.float32),
            pltpu.VMEM((RN,), jnp.float32),
            pltpu.SemaphoreType.DMA,
            pltpu.SemaphoreType.DMA,
            pltpu.SemaphoreType.DMA,
            pltpu.SemaphoreType.DMA,
        ],
        compiler_params=pltpu.CompilerParams(needs_layout_passes=False),
    )
    def k(in_hbm, sel_hbm, out_hbm, sel_v, c_v, in0, in1, ob0, ob1,
          si0, si1, so0, so1):
        cid = lax.axis_index("c")
        sid = lax.axis_index("s")
        wid = sid * 2 + cid
        base = wid * (ROWS_PER_WORKER * DIM)

        def in_win(g):
            return in_hbm.at[pl.ds(pl.multiple_of(base + g * RN, 8), RN)]

        def out_win(g):
            return out_hbm.at[pl.ds(pl.multiple_of(base + g * RN, 8), RN)]

        pltpu.async_copy(in_win(0), in0, si0)
        pltpu.async_copy(in_win(1), in1, si1)

        pltpu.sync_copy(sel_hbm, sel_v)

        @plsc.parallel_loop(0, NSEG, unroll=8)
        def _(j):
            off = pl.multiple_of(j * LANES, LANES)
            seg = sel_v[pl.ds(off, LANES)]
            c_v[pl.ds(off, LANES)] = plsc.load_gather(sel_v, [seg])

        def pair(t, carry):
            for parity, inb, isem, outb, osem in (
                (0, in0, si0, ob0, so0),
                (1, in1, si1, ob1, so1),
            ):
                g = 2 * t + parity
                pltpu.make_async_copy(in_win(0), inb, isem).wait()

                @pl.when(t > 0)
                def _():
                    pltpu.make_async_copy(outb, out_win(0), osem).wait()

                @plsc.parallel_loop(0, NSEG, unroll=4)
                def _(j):
                    off = pl.multiple_of(j * LANES, LANES)
                    idx = c_v[pl.ds(off, LANES)]
                    for r in range(R):
                        vals = plsc.load_gather(inb, [idx + r * DIM])
                        outb[pl.ds(off + r * DIM, LANES)] = vals

                pltpu.async_copy(outb, out_win(g), osem)

                @pl.when(t < TPAIR - 1)
                def _():
                    pltpu.async_copy(in_win(g + 2), inb, isem)

            return carry

        lax.fori_loop(0, TPAIR, pair, 0)

        pltpu.make_async_copy(ob0, out_win(0), so0).wait()
        pltpu.make_async_copy(ob1, out_win(0), so1).wait()

    return k(inp_flat, sel)


def kernel(input, forward_indices, backward_indices, shifts):
    ident = jnp.arange(DIM, dtype=jnp.int32)
    sel = jnp.where(
        shifts > 0,
        forward_indices,
        jnp.where(shifts < 0, backward_indices, ident),
    )
    out = _sc_permute(jnp.reshape(input, (-1,)), sel)
    return jnp.reshape(out, (BATCH, DIM))


# speedup vs baseline: 2.3589x; 1.5161x over previous
import functools

import jax
import jax.numpy as jnp
from jax import lax
from jax.experimental import pallas as pl
from jax.experimental.pallas import tpu as pltpu
from jax.experimental.pallas import tpu_sc as plsc

BATCH = 4096
DIM = 10000
LANES = 16
NUM_WORKERS = 32
ROWS_PER_WORKER = BATCH // NUM_WORKERS
NSEG = DIM // LANES
R = 2
CHUNKS = ROWS_PER_WORKER // R
TPAIR = CHUNKS // 2


def _sc_permute(inp, sel):
    mesh = plsc.VectorSubcoreMesh(core_axis_name="c", subcore_axis_name="s")

    @functools.partial(
        pl.kernel,
        mesh=mesh,
        out_type=jax.ShapeDtypeStruct((BATCH, DIM), jnp.float32),
        scratch_types=[
            pltpu.VMEM((DIM,), jnp.int32),
            pltpu.VMEM((DIM,), jnp.int32),
            pltpu.VMEM((R, DIM), jnp.float32),
            pltpu.VMEM((R, DIM), jnp.float32),
            pltpu.VMEM((R, DIM), jnp.float32),
            pltpu.VMEM((R, DIM), jnp.float32),
            pltpu.SemaphoreType.DMA,
            pltpu.SemaphoreType.DMA,
            pltpu.SemaphoreType.DMA,
            pltpu.SemaphoreType.DMA,
        ],
        compiler_params=pltpu.CompilerParams(needs_layout_passes=False),
    )
    def k(in_hbm, sel_hbm, out_hbm, sel_v, c_v, in0, in1, ob0, ob1,
          si0, si1, so0, so1):
        cid = lax.axis_index("c")
        sid = lax.axis_index("s")
        wid = sid * 2 + cid
        base = wid * ROWS_PER_WORKER

        def in_win(g):
            return in_hbm.at[pl.ds(base + g * R, R)]

        def out_win(g):
            return out_hbm.at[pl.ds(base + g * R, R)]

        pltpu.async_copy(in_win(0), in0, si0)
        pltpu.async_copy(in_win(1), in1, si1)

        pltpu.sync_copy(sel_hbm, sel_v)

        @plsc.parallel_loop(0, NSEG, unroll=8)
        def _(j):
            off = pl.multiple_of(j * LANES, LANES)
            seg = sel_v[pl.ds(off, LANES)]
            c_v[pl.ds(off, LANES)] = plsc.load_gather(sel_v, [seg])

        row_ids = [jnp.full((LANES,), r, jnp.int32) for r in range(R)]

        def pair(t, carry):
            for parity, inb, isem, outb, osem in (
                (0, in0, si0, ob0, so0),
                (1, in1, si1, ob1, so1),
            ):
                g = 2 * t + parity
                pltpu.make_async_copy(in_win(0), inb, isem).wait()

                @pl.when(t > 0)
                def _():
                    pltpu.make_async_copy(outb, out_win(0), osem).wait()

                @plsc.parallel_loop(0, NSEG, unroll=4)
                def _(j):
                    off = pl.multiple_of(j * LANES, LANES)
                    idx = c_v[pl.ds(off, LANES)]
                    for r in range(R):
                        vals = plsc.load_gather(inb, [row_ids[r], idx])
                        outb[r, pl.ds(off, LANES)] = vals

                pltpu.async_copy(outb, out_win(g), osem)

                @pl.when(t < TPAIR - 1)
                def _():
                    pltpu.async_copy(in_win(g + 2), inb, isem)

            return carry

        lax.fori_loop(0, TPAIR, pair, 0)

        pltpu.make_async_copy(ob0, out_win(0), so0).wait()
        pltpu.make_async_copy(ob1, out_win(0), so1).wait()

    return k(inp, sel)


def kernel(input, forward_indices, backward_indices, shifts):
    ident = jnp.arange(DIM, dtype=jnp.int32)
    sel = jnp.where(
        shifts > 0,
        forward_indices,
        jnp.where(shifts < 0, backward_indices, ident),
    )
    return _sc_permute(input, sel)


# speedup vs baseline: 6.2797x; 2.6621x over previous
import functools

import jax
import jax.numpy as jnp
from jax import lax
from jax.experimental import pallas as pl
from jax.experimental.pallas import tpu as pltpu
from jax.experimental.pallas import tpu_sc as plsc

BATCH = 4096
DIM = 10000
LANES = 16
NUM_WORKERS = 32
NSEG = DIM // LANES
G = 8
NGROUPS = DIM // G
MAXP = NGROUPS // NUM_WORKERS + 1
TSTEPS = (MAXP + 2 + 2) // 3


def _sc_permute(inp_t, sel):
    mesh = plsc.VectorSubcoreMesh(core_axis_name="c", subcore_axis_name="s")

    @functools.partial(
        pl.kernel,
        mesh=mesh,
        out_type=jax.ShapeDtypeStruct((DIM, BATCH), jnp.float32),
        scratch_types=[
            pltpu.VMEM((DIM,), jnp.int32),
            pltpu.VMEM((DIM,), jnp.int32),
            pltpu.VMEM((G, BATCH), jnp.float32),
            pltpu.VMEM((G, BATCH), jnp.float32),
            pltpu.VMEM((G, BATCH), jnp.float32),
            pltpu.SemaphoreType.DMA,
            pltpu.SemaphoreType.DMA,
            pltpu.SemaphoreType.DMA,
            pltpu.SemaphoreType.DMA,
            pltpu.SemaphoreType.DMA,
            pltpu.SemaphoreType.DMA,
        ],
        compiler_params=pltpu.CompilerParams(needs_layout_passes=False),
    )
    def k(in_hbm, sel_hbm, out_hbm, sel_v, c_v, b0, b1, b2,
          g0, g1, g2, w0, w1, w2):
        cid = lax.axis_index("c")
        sid = lax.axis_index("s")
        wid = sid * 2 + cid
        nvalid = jnp.where(wid < NGROUPS % NUM_WORKERS, MAXP, MAXP - 1)

        pltpu.sync_copy(sel_hbm, sel_v)

        @plsc.parallel_loop(0, NSEG, unroll=8)
        def _(j):
            off = pl.multiple_of(j * LANES, LANES)
            seg = sel_v[pl.ds(off, LANES)]
            c_v[pl.ds(off, LANES)] = plsc.load_gather(sel_v, [seg])

        def step(p, carry):
            grp = wid + NUM_WORKERS * p

            @pl.when(p < nvalid)
            def _():
                idx = c_v.at[pl.ds(pl.multiple_of(grp * G, G), G)]
                pltpu.async_copy(in_hbm.at[idx], b0, g0).wait()
                pltpu.async_copy(
                    b0, out_hbm.at[pl.ds(pl.multiple_of(grp * G, G), G)],
                    w0).wait()

            return carry

        lax.fori_loop(0, MAXP, step, 0)

    return k(inp_t, sel)


def kernel(input, forward_indices, backward_indices, shifts):
    ident = jnp.arange(DIM, dtype=jnp.int32)
    sel = jnp.where(
        shifts > 0,
        forward_indices,
        jnp.where(shifts < 0, backward_indices, ident),
    )
    out_t = _sc_permute(jnp.transpose(input), sel)
    return jnp.transpose(out_t)


# speedup vs baseline: 7.6750x; 1.2222x over previous
import functools

import jax
import jax.numpy as jnp
from jax import lax
from jax.experimental import pallas as pl
from jax.experimental.pallas import tpu as pltpu
from jax.experimental.pallas import tpu_sc as plsc

BATCH = 4096
DIM = 10000
LANES = 16
NUM_WORKERS = 32
NSEG = DIM // LANES
G = 8
NGROUPS = DIM // G
MAXP = NGROUPS // NUM_WORKERS + 1
TSTEPS = (MAXP + 2 + 2) // 3


def _sc_permute(inp_t, sel):
    mesh = plsc.VectorSubcoreMesh(core_axis_name="c", subcore_axis_name="s")

    @functools.partial(
        pl.kernel,
        mesh=mesh,
        out_type=jax.ShapeDtypeStruct((DIM, BATCH), jnp.float32),
        scratch_types=[
            pltpu.VMEM((DIM,), jnp.int32),
            pltpu.VMEM((DIM,), jnp.int32),
            pltpu.VMEM((G, BATCH), jnp.float32),
            pltpu.VMEM((G, BATCH), jnp.float32),
            pltpu.SemaphoreType.DMA,
            pltpu.SemaphoreType.DMA,
            pltpu.SemaphoreType.DMA,
            pltpu.SemaphoreType.DMA,
        ],
        compiler_params=pltpu.CompilerParams(needs_layout_passes=False),
    )
    def k(in_hbm, sel_hbm, out_hbm, sel_v, c_v, b0, b1, g0, g1, w0, w1):
        cid = lax.axis_index("c")
        sid = lax.axis_index("s")
        wid = sid * 2 + cid
        nvalid = jnp.where(wid < NGROUPS % NUM_WORKERS, MAXP, MAXP - 1)

        pltpu.sync_copy(sel_hbm, sel_v)

        @plsc.parallel_loop(0, NSEG, unroll=8)
        def _(j):
            off = pl.multiple_of(j * LANES, LANES)
            seg = sel_v[pl.ds(off, LANES)]
            c_v[pl.ds(off, LANES)] = plsc.load_gather(sel_v, [seg])

        def fire_gather(p, buf, gsem):
            grp = wid + NUM_WORKERS * p
            idx = c_v.at[pl.ds(pl.multiple_of(grp * G, G), G)]
            pltpu.async_copy(in_hbm.at[idx], buf, gsem)

        fire_gather(0, b0, g0)

        def step(t, carry):
            for par, buf, gsem, obuf, ogsem, wsem in (
                (0, b0, g0, b1, g1, w0),
                (1, b1, g1, b0, g0, w1),
            ):
                p = 2 * t + par
                grp = wid + NUM_WORKERS * p

                @pl.when(p + 1 < nvalid)
                def _():
                    fire_gather(p + 1, obuf, ogsem)

                @pl.when(p < nvalid)
                def _():
                    pltpu.make_async_copy(in_hbm.at[pl.ds(0, G)], buf,
                                          gsem).wait()
                    pltpu.async_copy(
                        buf, out_hbm.at[pl.ds(pl.multiple_of(grp * G, G), G)],
                        wsem).wait()

            return carry

        lax.fori_loop(0, MAXP // 2, step, 0)

    return k(inp_t, sel)


def kernel(input, forward_indices, backward_indices, shifts):
    ident = jnp.arange(DIM, dtype=jnp.int32)
    sel = jnp.where(
        shifts > 0,
        forward_indices,
        jnp.where(shifts < 0, backward_indices, ident),
    )
    out_t = _sc_permute(jnp.transpose(input), sel)
    return jnp.transpose(out_t)
